# baseline (speedup 1.0000x reference)
import functools

import jax
import jax.numpy as jnp
from jax import lax
from jax.experimental import pallas as pl
from jax.experimental.pallas import tpu as pltpu
from jax.experimental.pallas import tpu_sc as plsc

_NC, _NS = 2, 16
_NW = _NC * _NS
_SLICES_PER_W = 3072 // _NW

_GDN = lax.GatherDimensionNumbers(
    offset_dims=(), collapsed_slice_dims=(0,), start_index_map=(0,))


def _gat(v, idx):
    return lax.gather(v, idx[:, None], _GDN, (1,),
                      mode=lax.GatherScatterMode.PROMISE_IN_BOUNDS)


def _sc_body(x_hbm, idx_hbm, out_hbm, idx_v, in_v, out_v):
    wid = lax.axis_index("s") * _NC + lax.axis_index("c")
    pltpu.sync_copy(idx_hbm, idx_v)
    cols = [idx_v[j, :] for j in range(4)]
    mask0 = cols[0] < 16
    mask1 = cols[1] < 16
    a_lo = jnp.where(mask0, cols[0], 0)
    a_hi = jnp.where(mask0, 0, cols[0] - 16)
    b_lo = jnp.where(mask1, cols[1], 0)
    b_hi = jnp.where(mask1, 0, cols[1] - 16)

    def do_slice(i, carry):
        t = wid * _SLICES_PER_W + i
        b = t // 192
        c = t % 192
        pltpu.sync_copy(x_hbm.at[b, c], in_v)

        def row_body(r, cc):
            v0 = in_v[r, 0:16]
            v1 = in_v[r, 16:32]
            v2 = in_v[r, 32:48]
            v3 = in_v[r, 48:64]
            u0 = jnp.maximum(v0, v2)
            u1 = jnp.maximum(v1, v3)
            s0 = jnp.where(mask0, _gat(u0, a_lo), _gat(u1, a_hi))
            s1 = jnp.where(mask1, _gat(u0, b_lo), _gat(u1, b_hi))
            out_v[r, :] = jnp.maximum(s0, s1)
            return cc

        lax.fori_loop(0, 196, row_body, 0, unroll=4)
        pltpu.sync_copy(out_v, out_hbm.at[b, c])
        return carry

    lax.fori_loop(0, _SLICES_PER_W, do_slice, 0)


def kernel(x, indices):
    b, c, s, g = x.shape
    mesh = plsc.VectorSubcoreMesh(core_axis_name="c", subcore_axis_name="s")
    run = functools.partial(
        pl.kernel,
        out_type=jax.ShapeDtypeStruct((b, c, s, 16), x.dtype),
        mesh=mesh,
        scratch_types=[
            pltpu.VMEM((4, 16), jnp.int32),
            pltpu.VMEM((s, g), jnp.float32),
            pltpu.VMEM((s, 16), jnp.float32),
        ],
        compiler_params=pltpu.CompilerParams(needs_layout_passes=False),
    )(_sc_body)
    return run(x, indices.astype(jnp.int32))
